# baseline (speedup 1.0000x reference)
import functools

import jax
import jax.numpy as jnp
from jax import lax
from jax.experimental import pallas as pl
from jax.experimental.pallas import tpu as pltpu
from jax.experimental.pallas import tpu_sc as plsc

_LANES = 16


def _make_sc_kernel(N, HID, D, NC, NS, C):
    NW = NC * NS
    rows_pw = N // NW
    n_chunks = rows_pw // C
    idx_pw = 4 * rows_pw
    mesh = plsc.VectorSubcoreMesh(core_axis_name="c", subcore_axis_name="s")

    @functools.partial(
        pl.kernel,
        mesh=mesh,
        out_type=jax.ShapeDtypeStruct((N, HID), jnp.float32),
        scratch_types=[
            pltpu.VMEM((idx_pw,), jnp.int32),
            pltpu.VMEM((4, C, HID), jnp.float32),
            pltpu.VMEM((2, 4 * C, D), jnp.float32),
            pltpu.SemaphoreType.DMA,
            pltpu.SemaphoreType.DMA,
            pltpu.SemaphoreType.DMA,
            pltpu.SemaphoreType.DMA,
        ],
    )
    def k(x_hbm, idx_hbm, tab_hbm, out_hbm, idx_v, x_v, e_v, si0, si1, so0, so1):
        s_in = (si0, si1)
        s_out = (so0, so1)
        wid = lax.axis_index("s") * NC + lax.axis_index("c")
        base = wid * rows_pw
        pltpu.sync_copy(idx_hbm.at[pl.ds(wid * idx_pw, idx_pw)], idx_v)

        def in_copies(g, xs, es):
            r0 = base + g * C
            sem = s_in[es]
            return (
                pltpu.make_async_copy(x_hbm.at[pl.ds(r0, C), :], x_v.at[xs], sem),
                pltpu.make_async_copy(
                    tab_hbm.at[idx_v.at[pl.ds(g * (4 * C), 4 * C)]], e_v.at[es], sem
                ),
            )

        def out_copy(g, xs, es):
            r0 = base + g * C
            return pltpu.make_async_copy(
                x_v.at[xs], out_hbm.at[pl.ds(r0, C), :], s_out[es]
            )

        def add_chunk(xs, es):
            def row(c, carry):
                for j in range(4):
                    for t in range(D // _LANES):
                        plsc.addupdate(
                            x_v.at[xs, c, pl.ds(j * D + t * _LANES, _LANES)],
                            e_v[es, j * C + c, pl.ds(t * _LANES, _LANES)],
                        )
                return carry

            lax.fori_loop(0, C, row, 0, unroll=False)

        def super_chunk(g2, carry):
            for u in range(4):
                g = g2 * 4 + u
                b = u % 2

                @pl.when(g < n_chunks - 1)
                def _fire_next():
                    for d in in_copies(g + 1, (u + 1) % 4, 1 - b):
                        d.start()

                for d in in_copies(g, u, b):
                    d.wait()
                add_chunk(u, b)

                @pl.when(g >= 1)
                def _drain_prev_out():
                    out_copy(g - 1, (u + 3) % 4, 1 - b).wait()

                out_copy(g, u, b).start()
            return carry

        for d in in_copies(0, 0, 0):
            d.start()
        lax.fori_loop(0, n_chunks // 4, super_chunk, 0, unroll=False)
        out_copy(n_chunks - 1, 3, 1).wait()

    return k


def _make_tc_kernel(M, HID, D, VT, BR):
    def body(idx_ref, x_ref, tab_ref, o_ref):
        tabv = tab_ref[...]
        parts = []
        for j in range(4):
            ids = idx_ref[0, j]
            oh = (
                ids[:, None]
                == lax.broadcasted_iota(jnp.int32, (BR, VT), 1)
            ).astype(jnp.float32)
            parts.append(
                lax.dot_general(
                    oh,
                    tabv,
                    (((1,), (0,)), ((), ())),
                    preferred_element_type=jnp.float32,
                )
            )
        o_ref[...] = x_ref[...] + jnp.concatenate(parts, axis=1)

    return pl.pallas_call(
        body,
        grid=(M // BR,),
        in_specs=[
            pl.BlockSpec((1, 8, BR), lambda i: (i, 0, 0)),
            pl.BlockSpec((BR, HID), lambda i: (i, 0)),
            pl.BlockSpec((VT, D), lambda i: (0, 0)),
        ],
        out_specs=pl.BlockSpec((BR, HID), lambda i: (i, 0)),
        out_shape=jax.ShapeDtypeStruct((M, HID), jnp.float32),
    )


def kernel(x, chan_ind, spat_ind, embed_channel, embed_spatial):
    N, HID = x.shape
    VC, D = embed_channel.shape
    VT = VC + embed_spatial.shape[0]
    C = 16
    S = N // 2
    BR = 256
    tab = jnp.concatenate([embed_channel, embed_spatial], axis=0)
    idx_all = jnp.concatenate(
        [chan_ind.astype(jnp.int32), spat_ind.astype(jnp.int32) + VC], axis=1
    ).T
    idx_sc = (
        idx_all[:, :S].reshape(4, S // C, C).transpose(1, 0, 2).reshape(S * 4)
    )
    idx_tc = jnp.concatenate([idx_all[:, S:], idx_all[:, S:]], axis=0)
    idx_tc = idx_tc.reshape(8, (N - S) // BR, BR).transpose(1, 0, 2)
    info = plsc.get_sparse_core_info()
    sc_k = _make_sc_kernel(S, HID, D, info.num_cores, info.num_subcores, C)
    tc_k = _make_tc_kernel(N - S, HID, D, VT, BR)
    out_sc = sc_k(x[:S], idx_sc, tab)
    out_tc = tc_k(idx_tc, x[S:], tab)
    return jnp.concatenate([out_sc, out_tc], axis=0)


# speedup vs baseline: 1.3045x; 1.3045x over previous
import functools

import jax
import jax.numpy as jnp
from jax import lax
from jax.experimental import pallas as pl
from jax.experimental.pallas import tpu as pltpu
from jax.experimental.pallas import tpu_sc as plsc

_LANES = 16


def _make_sc_kernel(N, HID, D, NC, NS, C):
    NW = NC * NS
    rows_pw = N // NW
    n_chunks = rows_pw // C
    mesh = plsc.VectorSubcoreMesh(core_axis_name="c", subcore_axis_name="s")

    @functools.partial(
        pl.kernel,
        mesh=mesh,
        out_type=jax.ShapeDtypeStruct((N, HID), jnp.float32),
        scratch_types=[
            pltpu.VMEM((4, rows_pw), jnp.int32),
            pltpu.VMEM((4, C, HID), jnp.float32),
            pltpu.VMEM((2, 4, C, D), jnp.float32),
            pltpu.SemaphoreType.DMA,
            pltpu.SemaphoreType.DMA,
            pltpu.SemaphoreType.DMA,
            pltpu.SemaphoreType.DMA,
        ],
    )
    def k(x_hbm, idx_hbm, tc_hbm, ts_hbm, out_hbm, idx_v, x_v, e_v, si0, si1, so0, so1):
        s_in = (si0, si1)
        s_out = (so0, so1)
        wid = lax.axis_index("s") * NC + lax.axis_index("c")
        base = wid * rows_pw
        pltpu.sync_copy(idx_hbm.at[:, pl.ds(base, rows_pw)], idx_v)

        def in_copies(g, xs, es):
            r0 = base + g * C
            c0 = g * C
            sem = s_in[es]
            return (
                pltpu.make_async_copy(x_hbm.at[pl.ds(r0, C), :], x_v.at[xs], sem),
                pltpu.make_async_copy(tc_hbm.at[idx_v.at[0, pl.ds(c0, C)]], e_v.at[es, 0], sem),
                pltpu.make_async_copy(tc_hbm.at[idx_v.at[1, pl.ds(c0, C)]], e_v.at[es, 1], sem),
                pltpu.make_async_copy(ts_hbm.at[idx_v.at[2, pl.ds(c0, C)]], e_v.at[es, 2], sem),
                pltpu.make_async_copy(ts_hbm.at[idx_v.at[3, pl.ds(c0, C)]], e_v.at[es, 3], sem),
            )

        def out_copy(g, xs, es):
            r0 = base + g * C
            return pltpu.make_async_copy(
                x_v.at[xs], out_hbm.at[pl.ds(r0, C), :], s_out[es]
            )

        def add_chunk(xs, es):
            def row(c, carry):
                for j in range(4):
                    for t in range(D // _LANES):
                        plsc.addupdate(
                            x_v.at[xs, c, pl.ds(j * D + t * _LANES, _LANES)],
                            e_v[es, j, c, pl.ds(t * _LANES, _LANES)],
                        )
                return carry

            lax.fori_loop(0, C, row, 0, unroll=False)

        def super_chunk(g2, carry):
            for u in range(4):
                g = g2 * 4 + u
                b = u % 2

                @pl.when(g < n_chunks - 1)
                def _fire_next():
                    for d in in_copies(g + 1, (u + 1) % 4, 1 - b):
                        d.start()

                for d in in_copies(g, u, b):
                    d.wait()
                add_chunk(u, b)

                @pl.when(g >= 1)
                def _drain_prev_out():
                    out_copy(g - 1, (u + 3) % 4, 1 - b).wait()

                out_copy(g, u, b).start()
            return carry

        for d in in_copies(0, 0, 0):
            d.start()
        lax.fori_loop(0, n_chunks // 4, super_chunk, 0, unroll=False)
        out_copy(n_chunks - 1, 3, 1).wait()

    return k


def kernel(x, chan_ind, spat_ind, embed_channel, embed_spatial):
    N, HID = x.shape
    D = embed_channel.shape[1]
    idx_all = jnp.concatenate(
        [chan_ind.astype(jnp.int32), spat_ind.astype(jnp.int32)], axis=1
    ).T
    info = plsc.get_sparse_core_info()
    k = _make_sc_kernel(N, HID, D, info.num_cores, info.num_subcores, 16)
    return k(x, idx_all, embed_channel, embed_spatial)
